# baseline (speedup 1.0000x reference)
import functools
import math

import jax
import jax.numpy as jnp
from jax import lax
from jax.experimental import pallas as pl
from jax.experimental.pallas import tpu as pltpu
from jax.experimental.pallas import tpu_sc as plsc

SCALE = 32.0
ALPHA = 0.2
W = 2000
NEG = -3.0e38


def _sc_row_stats(score_flat, y32, batch, num_cls):
    info = plsc.get_sparse_core_info()
    nw = info.num_cores * info.num_subcores
    rpw = batch // nw
    ngroups = rpw // 16
    nch = num_cls // W
    vpc = W // 16

    mesh = plsc.VectorSubcoreMesh(core_axis_name="c", subcore_axis_name="s")

    @functools.partial(
        pl.kernel,
        mesh=mesh,
        out_type=[
            jax.ShapeDtypeStruct((batch * 16,), jnp.float32),
            jax.ShapeDtypeStruct((batch * 16,), jnp.float32),
            jax.ShapeDtypeStruct((batch,), jnp.float32),
        ],
        scratch_types=(
            [pltpu.VMEM((W,), jnp.float32) for _ in range(32)]
            + [
                pltpu.VMEM((256,), jnp.float32),
                pltpu.VMEM((256,), jnp.float32),
                pltpu.VMEM((16,), jnp.float32),
                pltpu.VMEM((16,), jnp.int32),
                pltpu.SemaphoreType.DMA,
                pltpu.SemaphoreType.DMA,
                pltpu.SemaphoreType.DMA,
            ]
        ),
    )
    def k(score_hbm, y_hbm, m_out, s_out, t_out, *scr):
        buf0 = scr[0:16]
        buf1 = scr[16:32]
        mst, sst, tv, yv, sem0, sem1, semg = scr[32:]
        wid = lax.axis_index("s") * info.num_cores + lax.axis_index("c")
        row0 = wid * rpw

        def fire(bufs, sem, rows16, c):
            for r in range(16):
                src = score_hbm.at[pl.ds((rows16 + r) * num_cls + c * W, W)]
                pltpu.make_async_copy(src, bufs[r], sem).start()

        def drain(bufs, sem):
            for r in range(16):
                pltpu.make_async_copy(
                    score_hbm.at[pl.ds(0, W)], bufs[r], sem
                ).wait()

        def chunk_update(bufs, carry):
            out = []
            for r in range(16):
                def one(j, mk, r=r):
                    mr, sr = mk
                    v = bufs[r][pl.ds(16 * j, 16)]
                    mn = jnp.maximum(mr, v)
                    sr = sr * jnp.exp(SCALE * (mr - mn)) + jnp.exp(
                        SCALE * (v - mn)
                    )
                    return mn, sr

                out.append(
                    lax.fori_loop(0, vpc, one, (carry[0][r], carry[1][r]),
                                  unroll=8)
                )
            return [o[0] for o in out], [o[1] for o in out]

        def group_body(g, _):
            rows16 = row0 + g * 16
            fire(buf0, sem0, rows16, 0)
            fire(buf1, sem1, rows16, 1)

            def chunk_pair(i, carry):
                c0 = 2 * i
                drain(buf0, sem0)
                carry = chunk_update(buf0, carry)

                @pl.when(c0 + 2 < nch)
                def _():
                    fire(buf0, sem0, rows16, c0 + 2)

                drain(buf1, sem1)
                carry = chunk_update(buf1, carry)

                @pl.when(c0 + 3 < nch)
                def _():
                    fire(buf1, sem1, rows16, c0 + 3)

                return carry

            init = ([jnp.full((16,), NEG, jnp.float32) for _ in range(16)],
                    [jnp.zeros((16,), jnp.float32) for _ in range(16)])
            ms, ss = lax.fori_loop(0, nch // 2, chunk_pair, init)

            for r in range(16):
                mst[pl.ds(16 * r, 16)] = ms[r]
                sst[pl.ds(16 * r, 16)] = ss[r]
            pltpu.sync_copy(mst, m_out.at[pl.ds(rows16 * 16, 256)])
            pltpu.sync_copy(sst, s_out.at[pl.ds(rows16 * 16, 256)])

            pltpu.sync_copy(y_hbm.at[pl.ds(rows16, 16)], yv)
            idx = (rows16 + lax.iota(jnp.int32, 16)) * num_cls + yv[...]
            pltpu.async_copy(score_hbm.at[idx], tv, semg).wait()
            pltpu.sync_copy(tv, t_out.at[pl.ds(rows16, 16)])
            return 0

        lax.fori_loop(0, ngroups, group_body, 0)

    return k(score_flat, y32)


def _combine(m, s, t, batch):
    corr = math.exp(-SCALE * ALPHA) - 1.0

    def body(m_ref, s_ref, t_ref, out_ref):
        m2 = m_ref[...] * SCALE
        mrow = jnp.max(m2, axis=1, keepdims=True)
        srow = jnp.sum(s_ref[...] * jnp.exp(m2 - mrow), axis=1, keepdims=True)
        tt = t_ref[...] * SCALE
        sc = srow + jnp.exp(tt - mrow) * corr
        out_ref[...] = mrow + jnp.log(sc) - tt + SCALE * ALPHA

    return pl.pallas_call(
        body,
        out_shape=jax.ShapeDtypeStruct((batch, 1), jnp.float32),
    )(m.reshape(batch, 16), s.reshape(batch, 16), t.reshape(batch, 1))


def kernel(score, y):
    batch, num_cls = score.shape
    y32 = jnp.asarray(y).reshape(-1).astype(jnp.int32)
    m, s, t = _sc_row_stats(score.reshape(-1), y32, batch, num_cls)
    return _combine(m, s, t, batch).reshape(-1)


# speedup vs baseline: 1.2000x; 1.2000x over previous
import functools
import math

import jax
import jax.numpy as jnp
from jax import lax
from jax.experimental import pallas as pl
from jax.experimental.pallas import tpu as pltpu
from jax.experimental.pallas import tpu_sc as plsc

SCALE = 32.0
ALPHA = 0.2
W = 10000
ACC = 5
NEG = -3.0e38


def _sc_row_stats(score_flat, y32, batch, num_cls):
    info = plsc.get_sparse_core_info()
    nw = info.num_cores * info.num_subcores
    rpw = batch // nw
    ngroups = rpw // 16
    nch = num_cls // W
    vpc = W // 16
    niter = vpc // ACC

    mesh = plsc.VectorSubcoreMesh(core_axis_name="c", subcore_axis_name="s")

    @functools.partial(
        pl.kernel,
        mesh=mesh,
        out_type=[
            jax.ShapeDtypeStruct((batch * 16,), jnp.float32),
            jax.ShapeDtypeStruct((batch * 16,), jnp.float32),
            jax.ShapeDtypeStruct((batch,), jnp.float32),
        ],
        scratch_types=[
            pltpu.VMEM((W,), jnp.float32),
            pltpu.VMEM((W,), jnp.float32),
            pltpu.VMEM((256,), jnp.float32),
            pltpu.VMEM((256,), jnp.float32),
            pltpu.VMEM((16,), jnp.float32),
            pltpu.VMEM((16,), jnp.int32),
            pltpu.SemaphoreType.DMA,
            pltpu.SemaphoreType.DMA,
            pltpu.SemaphoreType.DMA,
        ],
    )
    def k(score_hbm, y_hbm, m_out, s_out, t_out,
          buf0, buf1, mst, sst, tv, yv, sem0, sem1, semg):
        wid = lax.axis_index("s") * info.num_cores + lax.axis_index("c")
        row0 = wid * rpw

        def fire(buf, sem, row, c):
            src = score_hbm.at[pl.ds(row * num_cls + c * W, W)]
            pltpu.make_async_copy(src, buf, sem).start()

        def drain(buf, sem):
            pltpu.make_async_copy(
                score_hbm.at[pl.ds(0, W)], buf, sem
            ).wait()

        def chunk_fold(buf, carry):
            m16, s16 = carry

            def pa(j, accs):
                new = []
                for a in range(ACC):
                    v = buf[pl.ds(16 * (ACC * j + a), 16)]
                    new.append(jnp.maximum(accs[a], v))
                return tuple(new)

            accs = lax.fori_loop(
                0, niter, pa, tuple(jnp.full((16,), NEG, jnp.float32)
                                    for _ in range(ACC))
            )
            cm = accs[0]
            for a in range(1, ACC):
                cm = jnp.maximum(cm, accs[a])
            mn = jnp.maximum(m16, cm)
            s16 = s16 * jnp.exp(SCALE * (m16 - mn))

            def pb(j, sums):
                new = []
                for a in range(ACC):
                    v = buf[pl.ds(16 * (ACC * j + a), 16)]
                    new.append(sums[a] + jnp.exp(SCALE * (v - mn)))
                return tuple(new)

            sums = lax.fori_loop(
                0, niter, pb, tuple(jnp.zeros((16,), jnp.float32)
                                    for _ in range(ACC))
            )
            for a in range(ACC):
                s16 = s16 + sums[a]
            return mn, s16

        def group_body(g, _):
            rows16 = row0 + g * 16
            for r in range(16):
                row = rows16 + r
                fire(buf0, sem0, row, 0)
                fire(buf1, sem1, row, 1)

                def chunk_pair(i, carry, row=row):
                    c0 = 2 * i
                    drain(buf0, sem0)
                    carry = chunk_fold(buf0, carry)

                    @pl.when(c0 + 2 < nch)
                    def _():
                        fire(buf0, sem0, row, c0 + 2)

                    drain(buf1, sem1)
                    carry = chunk_fold(buf1, carry)

                    @pl.when(c0 + 3 < nch)
                    def _():
                        fire(buf1, sem1, row, c0 + 3)

                    return carry

                init = (jnp.full((16,), NEG, jnp.float32),
                        jnp.zeros((16,), jnp.float32))
                m16, s16 = lax.fori_loop(0, nch // 2, chunk_pair, init)
                mst[pl.ds(16 * r, 16)] = m16
                sst[pl.ds(16 * r, 16)] = s16

            pltpu.sync_copy(mst, m_out.at[pl.ds(rows16 * 16, 256)])
            pltpu.sync_copy(sst, s_out.at[pl.ds(rows16 * 16, 256)])

            pltpu.sync_copy(y_hbm.at[pl.ds(rows16, 16)], yv)
            idx = (rows16 + lax.iota(jnp.int32, 16)) * num_cls + yv[...]
            pltpu.async_copy(score_hbm.at[idx], tv, semg).wait()
            pltpu.sync_copy(tv, t_out.at[pl.ds(rows16, 16)])
            return 0

        lax.fori_loop(0, ngroups, group_body, 0)

    return k(score_flat, y32)


def _combine(m, s, t, batch):
    corr = math.exp(-SCALE * ALPHA) - 1.0

    def body(m_ref, s_ref, t_ref, out_ref):
        m2 = m_ref[...] * SCALE
        mrow = jnp.max(m2, axis=1, keepdims=True)
        srow = jnp.sum(s_ref[...] * jnp.exp(m2 - mrow), axis=1, keepdims=True)
        tt = t_ref[...] * SCALE
        sc = srow + jnp.exp(tt - mrow) * corr
        out_ref[...] = mrow + jnp.log(sc) - tt + SCALE * ALPHA

    return pl.pallas_call(
        body,
        out_shape=jax.ShapeDtypeStruct((batch, 1), jnp.float32),
    )(m.reshape(batch, 16), s.reshape(batch, 16), t.reshape(batch, 1))


def kernel(score, y):
    batch, num_cls = score.shape
    y32 = jnp.asarray(y).reshape(-1).astype(jnp.int32)
    m, s, t = _sc_row_stats(score.reshape(-1), y32, batch, num_cls)
    return _combine(m, s, t, batch).reshape(-1)


# speedup vs baseline: 1.2161x; 1.0134x over previous
import functools
import math

import jax
import jax.numpy as jnp
from jax import lax
from jax.experimental import pallas as pl
from jax.experimental.pallas import tpu as pltpu
from jax.experimental.pallas import tpu_sc as plsc

SCALE = 32.0
ALPHA = 0.2
W = 10000
ACC = 5
NEG = -3.0e38


def _sc_row_stats(score_flat, y32, batch, num_cls):
    info = plsc.get_sparse_core_info()
    nw = info.num_cores * info.num_subcores
    rpw = batch // nw
    ngroups = rpw // 16
    nch = num_cls // W
    vpc = W // 16
    niter = vpc // ACC

    mesh = plsc.VectorSubcoreMesh(core_axis_name="c", subcore_axis_name="s")

    @functools.partial(
        pl.kernel,
        mesh=mesh,
        out_type=[
            jax.ShapeDtypeStruct((batch * 16,), jnp.float32),
            jax.ShapeDtypeStruct((batch * 16,), jnp.float32),
            jax.ShapeDtypeStruct((batch,), jnp.float32),
        ],
        scratch_types=[
            pltpu.VMEM((W,), jnp.float32),
            pltpu.VMEM((W,), jnp.float32),
            pltpu.VMEM((256,), jnp.float32),
            pltpu.VMEM((256,), jnp.float32),
            pltpu.VMEM((16,), jnp.float32),
            pltpu.VMEM((16,), jnp.int32),
            pltpu.SemaphoreType.DMA,
            pltpu.SemaphoreType.DMA,
            pltpu.SemaphoreType.DMA,
        ],
    )
    def k(score_hbm, y_hbm, m_out, s_out, t_out,
          buf0, buf1, mst, sst, tv, yv, sem0, sem1, semg):
        wid = lax.axis_index("s") * info.num_cores + lax.axis_index("c")
        row0 = wid * rpw

        def fire(buf, sem, row, c):
            src = score_hbm.at[pl.ds(row * num_cls + c * W, W)]
            pltpu.make_async_copy(src, buf, sem).start()

        def drain(buf, sem):
            pltpu.make_async_copy(
                score_hbm.at[pl.ds(0, W)], buf, sem
            ).wait()

        def chunk_fold(buf, carry):
            m16, s16 = carry

            def pa(j, accs):
                new = []
                for a in range(ACC):
                    v = buf[pl.ds(16 * (ACC * j + a), 16)]
                    new.append(jnp.maximum(accs[a], v))
                return tuple(new)

            accs = lax.fori_loop(
                0, niter, pa, tuple(jnp.full((16,), NEG, jnp.float32)
                                    for _ in range(ACC)),
                unroll=5,
            )
            cm = accs[0]
            for a in range(1, ACC):
                cm = jnp.maximum(cm, accs[a])
            mn = jnp.maximum(m16, cm)
            s16 = s16 * jnp.exp(SCALE * (m16 - mn))

            def pb(j, sums):
                new = []
                for a in range(ACC):
                    v = buf[pl.ds(16 * (ACC * j + a), 16)]
                    new.append(sums[a] + jnp.exp(SCALE * (v - mn)))
                return tuple(new)

            sums = lax.fori_loop(
                0, niter, pb, tuple(jnp.zeros((16,), jnp.float32)
                                    for _ in range(ACC)),
                unroll=5,
            )
            for a in range(ACC):
                s16 = s16 + sums[a]
            return mn, s16

        def group_body(g, _):
            rows16 = row0 + g * 16
            for r in range(16):
                row = rows16 + r
                fire(buf0, sem0, row, 0)
                fire(buf1, sem1, row, 1)

                def chunk_pair(i, carry, row=row):
                    c0 = 2 * i
                    drain(buf0, sem0)
                    carry = chunk_fold(buf0, carry)

                    @pl.when(c0 + 2 < nch)
                    def _():
                        fire(buf0, sem0, row, c0 + 2)

                    drain(buf1, sem1)
                    carry = chunk_fold(buf1, carry)

                    @pl.when(c0 + 3 < nch)
                    def _():
                        fire(buf1, sem1, row, c0 + 3)

                    return carry

                init = (jnp.full((16,), NEG, jnp.float32),
                        jnp.zeros((16,), jnp.float32))
                m16, s16 = lax.fori_loop(0, nch // 2, chunk_pair, init)
                mst[pl.ds(16 * r, 16)] = m16
                sst[pl.ds(16 * r, 16)] = s16

            pltpu.sync_copy(mst, m_out.at[pl.ds(rows16 * 16, 256)])
            pltpu.sync_copy(sst, s_out.at[pl.ds(rows16 * 16, 256)])

            pltpu.sync_copy(y_hbm.at[pl.ds(rows16, 16)], yv)
            idx = (rows16 + lax.iota(jnp.int32, 16)) * num_cls + yv[...]
            pltpu.async_copy(score_hbm.at[idx], tv, semg).wait()
            pltpu.sync_copy(tv, t_out.at[pl.ds(rows16, 16)])
            return 0

        lax.fori_loop(0, ngroups, group_body, 0)

    return k(score_flat, y32)


def _combine(m, s, t, batch):
    corr = math.exp(-SCALE * ALPHA) - 1.0

    def body(m_ref, s_ref, t_ref, out_ref):
        m2 = m_ref[...] * SCALE
        mrow = jnp.max(m2, axis=1, keepdims=True)
        srow = jnp.sum(s_ref[...] * jnp.exp(m2 - mrow), axis=1, keepdims=True)
        tt = t_ref[...] * SCALE
        sc = srow + jnp.exp(tt - mrow) * corr
        out_ref[...] = mrow + jnp.log(sc) - tt + SCALE * ALPHA

    return pl.pallas_call(
        body,
        out_shape=jax.ShapeDtypeStruct((batch, 1), jnp.float32),
    )(m.reshape(batch, 16), s.reshape(batch, 16), t.reshape(batch, 1))


def kernel(score, y):
    batch, num_cls = score.shape
    y32 = jnp.asarray(y).reshape(-1).astype(jnp.int32)
    m, s, t = _sc_row_stats(score.reshape(-1), y32, batch, num_cls)
    return _combine(m, s, t, batch).reshape(-1)


# speedup vs baseline: 1.6923x; 1.3916x over previous
import functools
import math

import jax
import jax.numpy as jnp
from jax import lax
from jax.experimental import pallas as pl
from jax.experimental.pallas import tpu as pltpu
from jax.experimental.pallas import tpu_sc as plsc

SCALE = 32.0
ALPHA = 0.2
TILES = 20
NEG = -3.0e38


def _sc_row_stats(score, y32, batch, num_cls):
    info = plsc.get_sparse_core_info()
    nw = info.num_cores * info.num_subcores
    rpw = batch // nw
    nrt = rpw // 8
    c0 = (num_cls // 128) * 128
    cw = TILES * 128
    nch = c0 // cw

    mesh = plsc.VectorSubcoreMesh(core_axis_name="c", subcore_axis_name="s")

    @functools.partial(
        pl.kernel,
        mesh=mesh,
        out_type=[
            jax.ShapeDtypeStruct((batch * 16,), jnp.float32),
            jax.ShapeDtypeStruct((batch * 16,), jnp.float32),
            jax.ShapeDtypeStruct((batch * 16,), jnp.float32),
        ],
        scratch_types=[
            pltpu.VMEM((TILES, 8, 128), jnp.float32),
            pltpu.VMEM((TILES, 8, 128), jnp.float32),
            pltpu.VMEM((128,), jnp.float32),
            pltpu.VMEM((128,), jnp.float32),
            pltpu.VMEM((128,), jnp.float32),
            pltpu.VMEM((32,), jnp.int32),
            pltpu.SemaphoreType.DMA,
            pltpu.SemaphoreType.DMA,
        ],
    )
    def k(score_hbm, y_hbm, m_out, s_out, t_out,
          buf0, buf1, mst, sst, tst, yv, sem0, sem1):
        wid = lax.axis_index("s") * info.num_cores + lax.axis_index("c")
        row0 = wid * rpw
        lanes = lax.iota(jnp.int32, 16)

        pltpu.sync_copy(y_hbm.at[pl.ds(row0, rpw)], yv)

        def fire(buf, sem, rtrow, c):
            for t in range(TILES):
                src = score_hbm.at[pl.ds(rtrow, 8), pl.ds(c * cw + t * 128, 128)]
                pltpu.make_async_copy(src, buf.at[t], sem).start()

        def drain(buf, sem):
            for t in range(TILES):
                pltpu.make_async_copy(
                    score_hbm.at[pl.ds(0, 8), pl.ds(0, 128)], buf.at[t], sem
                ).wait()

        def chunk_fold(buf, c, carry, yrows):
            ms, ss, ts = carry
            new_m, new_s, new_t = [], [], []
            for r in range(8):
                m16, s16, t16 = ms[r], ss[r], ts[r]

                def pa(t, accs, r=r):
                    new = []
                    for jj in range(8):
                        v = buf[t, r, pl.ds(16 * jj, 16)]
                        new.append(jnp.maximum(accs[jj], v))
                    return tuple(new)

                accs = lax.fori_loop(
                    0, TILES, pa,
                    tuple(jnp.full((16,), NEG, jnp.float32) for _ in range(8)),
                )
                cm = accs[0]
                for jj in range(1, 8):
                    cm = jnp.maximum(cm, accs[jj])
                mn = jnp.maximum(m16, cm)
                s16 = s16 * jnp.exp(SCALE * (m16 - mn))
                yr16 = yrows[r]

                def pb(t, sums, r=r, yr16=yr16, mn=mn):
                    acc, tacc = sums
                    new = []
                    for jj in range(8):
                        v = buf[t, r, pl.ds(16 * jj, 16)]
                        col = c * cw + t * 128 + 16 * jj + lanes
                        tacc = tacc + jnp.where(col == yr16, v, 0.0)
                        new.append(acc[jj] + jnp.exp(SCALE * (v - mn)))
                    return tuple(new), tacc

                sums, t16 = lax.fori_loop(
                    0, TILES, pb,
                    (tuple(jnp.zeros((16,), jnp.float32) for _ in range(8)),
                     t16),
                )
                for jj in range(8):
                    s16 = s16 + sums[jj]
                new_m.append(mn)
                new_s.append(s16)
                new_t.append(t16)
            return new_m, new_s, new_t

        def rowtile_body(rt, _):
            rtrow = row0 + rt * 8
            yrows = []
            for r in range(8):
                idx = rt * 8 + r
                half = (idx // 16) * 16
                lane = idx - half
                ys16 = yv[pl.ds(half, 16)]
                yrows.append(
                    ys16.at[jnp.full((16,), lane, jnp.int32)].get(
                        mode="promise_in_bounds"
                    )
                )

            fire(buf0, sem0, rtrow, 0)
            fire(buf1, sem1, rtrow, 1)

            def chunk_pair(i, carry):
                c0i = 2 * i
                drain(buf0, sem0)
                carry = chunk_fold(buf0, c0i, carry, yrows)

                @pl.when(c0i + 2 < nch)
                def _():
                    fire(buf0, sem0, rtrow, c0i + 2)

                drain(buf1, sem1)
                carry = chunk_fold(buf1, c0i + 1, carry, yrows)

                @pl.when(c0i + 3 < nch)
                def _():
                    fire(buf1, sem1, rtrow, c0i + 3)

                return carry

            init = ([jnp.full((16,), NEG, jnp.float32) for _ in range(8)],
                    [jnp.zeros((16,), jnp.float32) for _ in range(8)],
                    [jnp.zeros((16,), jnp.float32) for _ in range(8)])
            carry = lax.fori_loop(0, nch // 2, chunk_pair, init)
            drain(buf0, sem0)
            ms, ss, ts = chunk_fold(buf0, nch - 1, carry, yrows)

            for r in range(8):
                mst[pl.ds(16 * r, 16)] = ms[r]
                sst[pl.ds(16 * r, 16)] = ss[r]
                tst[pl.ds(16 * r, 16)] = ts[r]
            pltpu.sync_copy(mst, m_out.at[pl.ds(rtrow * 16, 128)])
            pltpu.sync_copy(sst, s_out.at[pl.ds(rtrow * 16, 128)])
            pltpu.sync_copy(tst, t_out.at[pl.ds(rtrow * 16, 128)])
            return 0

        lax.fori_loop(0, nrt, rowtile_body, 0)

    return k(score, y32)


def _combine(m, s, t, tail, y32, batch, num_cls):
    corr = math.exp(-SCALE * ALPHA) - 1.0
    c0 = (num_cls // 128) * 128
    tw = num_cls - c0

    def body(m_ref, s_ref, t_ref, tail_ref, y_ref, out_ref):
        vt = tail_ref[...]
        cols = c0 + lax.broadcasted_iota(jnp.int32, vt.shape, 1)
        ymask = cols == y_ref[...]
        mtail = jnp.max(vt, axis=1, keepdims=True) * SCALE
        m2 = m_ref[...] * SCALE
        mrow = jnp.maximum(jnp.max(m2, axis=1, keepdims=True), mtail)
        srow = jnp.sum(s_ref[...] * jnp.exp(m2 - mrow), axis=1, keepdims=True)
        srow = srow + jnp.sum(
            jnp.exp(vt * SCALE - mrow), axis=1, keepdims=True
        )
        traw = jnp.sum(t_ref[...], axis=1, keepdims=True) + jnp.sum(
            jnp.where(ymask, vt, 0.0), axis=1, keepdims=True
        )
        tt = traw * SCALE
        sc = srow + jnp.exp(tt - mrow) * corr
        out_ref[...] = mrow + jnp.log(sc) - tt + SCALE * ALPHA

    return pl.pallas_call(
        body,
        out_shape=jax.ShapeDtypeStruct((batch, 1), jnp.float32),
    )(
        m.reshape(batch, 16),
        s.reshape(batch, 16),
        t.reshape(batch, 16),
        tail,
        y32.reshape(batch, 1),
    )


def kernel(score, y):
    batch, num_cls = score.shape
    y32 = jnp.asarray(y).reshape(-1).astype(jnp.int32)
    m, s, t = _sc_row_stats(score, y32, batch, num_cls)
    tail = score[:, (num_cls // 128) * 128 :]
    return _combine(m, s, t, tail, y32, batch, num_cls).reshape(-1)
